# baseline (speedup 1.0000x reference)
import functools

import jax
import jax.numpy as jnp
from jax import lax
from jax.experimental import pallas as pl
from jax.experimental.pallas import tpu as pltpu
from jax.experimental.pallas import tpu_sc as plsc

_NCLS = 3
_NC = 2
_NS = 16
_NW = _NC * _NS
_L = 16
_NSEG = 4
_CHUNK = 4096
_UNROLL = 4


def _focal_group(p, lbl, c, accl):
    t = lbl == (c + 1)
    z = jnp.where(t, -p, p)
    e = jnp.exp(-jnp.abs(p))
    inv = 1.0 / (1.0 + e)
    sig = jnp.where(z >= 0, inv, e * inv)
    y = e / (2.0 + e)
    y2 = y * y
    lg = y * (2.0 + y2 * (2.0 / 3.0 + y2 * (2.0 / 5.0)))
    sp = jnp.maximum(z, 0.0) + lg
    aw = jnp.where(t, 0.25, 0.75)
    return accl + aw * (sig * sig) * sp


def _focal_partial_body(n_seg, lbl_off, preds_hbm, labels_hbm, part_out,
                        pv0, pv1, lv0, lv1, stage_v, sem0, sem1):
    wid = lax.axis_index("s") * _NC + lax.axis_index("c")
    ppt = n_seg // _NW
    nchunks = ppt // _CHUNK
    bufs = ((pv0, lv0, sem0), (pv1, lv1, sem1))

    def start(ci, buf):
        pv, lv, sem = buf
        pbase = wid * ppt + ci * _CHUNK
        handles = []
        for c in range(_NCLS):
            handles.append(pltpu.async_copy(
                preds_hbm.at[pl.ds(c * n_seg + pbase, _CHUNK)],
                pv.at[pl.ds(c * _CHUNK, _CHUNK)], sem))
        handles.append(pltpu.async_copy(
            labels_hbm.at[pl.ds(lbl_off + pbase, _CHUNK)], lv, sem))
        return handles

    pending = {0: start(0, bufs[0])}

    accl = jnp.zeros((_L,), jnp.float32)
    accp = jnp.zeros((_L,), jnp.float32)
    for ci in range(nchunks):
        b = ci % 2
        pv, lv, _ = bufs[b]
        for h in pending.pop(ci):
            h.wait()
        if ci + 1 < nchunks:
            pending[ci + 1] = start(ci + 1, bufs[1 - b])

        def group_body(g, acc, pv=pv, lv=lv):
            accl, accp = acc
            for u in range(_UNROLL):
                off = (g * _UNROLL + u) * _L
                lbl = lv[pl.ds(off, _L)]
                accp = accp + jnp.where(lbl > 0, 1.0, 0.0)
                for c in range(_NCLS):
                    p = pv[pl.ds(c * _CHUNK + off, _L)]
                    accl = _focal_group(p, lbl, c, accl)
            return accl, accp

        accl, accp = lax.fori_loop(0, _CHUNK // (_L * _UNROLL), group_body,
                                   (accl, accp))

    stage_v[pl.ds(0, _L)] = accl
    stage_v[pl.ds(_L, _L)] = accp
    pltpu.sync_copy(stage_v.at[pl.ds(0, _L)],
                    part_out.at[pl.ds(wid * _L, _L)])
    pltpu.sync_copy(stage_v.at[pl.ds(_L, _L)],
                    part_out.at[pl.ds(_NW * _L + wid * _L, _L)])


def _reduce_tc_body(*refs):
    parts = refs[:-1]
    o_ref = refs[-1]
    s = jnp.zeros((), jnp.float32)
    q = jnp.zeros((), jnp.float32)
    for p_ref in parts:
        part = p_ref[...]
        s = s + jnp.sum(part[:4, :])
        q = q + jnp.sum(part[4:, :])
    o_ref[...] = jnp.reshape(s / jnp.maximum(q, 1.0), (1, 1))


@functools.lru_cache(maxsize=None)
def _build(n_points):
    n_seg = n_points // _NSEG
    mesh = plsc.VectorSubcoreMesh(core_axis_name="c", subcore_axis_name="s")
    seg_fns = []
    for s in range(_NSEG):
        seg_fns.append(functools.partial(
            pl.kernel,
            mesh=mesh,
            out_type=jax.ShapeDtypeStruct((2 * _NW * _L,), jnp.float32),
            scratch_types=[
                pltpu.VMEM((_CHUNK * _NCLS,), jnp.float32),
                pltpu.VMEM((_CHUNK * _NCLS,), jnp.float32),
                pltpu.VMEM((_CHUNK,), jnp.int32),
                pltpu.VMEM((_CHUNK,), jnp.int32),
                pltpu.VMEM((2 * _L,), jnp.float32),
                pltpu.SemaphoreType.DMA,
                pltpu.SemaphoreType.DMA,
            ],
        )(functools.partial(_focal_partial_body, n_seg, s * n_seg)))
    reduce_fn = pl.pallas_call(
        _reduce_tc_body,
        out_shape=jax.ShapeDtypeStruct((1, 1), jnp.float32),
    )
    return seg_fns, reduce_fn


@jax.jit
def kernel(point_cls_preds, point_cls_labels):
    n = point_cls_labels.shape[0]
    n_seg = n // _NSEG
    seg_fns, reduce_fn = _build(n)
    preds_t = point_cls_preds.T
    parts = []
    for s, fn in enumerate(seg_fns):
        seg = preds_t[:, s * n_seg:(s + 1) * n_seg].reshape(-1)
        parts.append(fn(seg, point_cls_labels))
    out = reduce_fn(*[p.reshape(8, _NW * _L // 4) for p in parts])
    return out[0, 0]


# speedup vs baseline: 1.2411x; 1.2411x over previous
import functools

import jax
import jax.numpy as jnp
from jax import lax
from jax.experimental import pallas as pl
from jax.experimental.pallas import tpu as pltpu
from jax.experimental.pallas import tpu_sc as plsc

_NCLS = 3
_NC = 2
_NS = 16
_NW = _NC * _NS
_L = 16
_NSEG = 3
_CHUNK = 4096
_UNROLL = 2
_SC_FRAC_NUM, _SC_FRAC_DEN = 3, 4
_TC_BS = 16384


def _focal_group(p, lbl, c, accl):
    t = lbl == (c + 1)
    z = jnp.where(t, -p, p)
    e = jnp.exp(-jnp.abs(p))
    inv = 1.0 / (1.0 + e)
    sig = jnp.where(z >= 0, inv, e * inv)
    y = e / (2.0 + e)
    y2 = y * y
    lg = y * (2.0 + y2 * (2.0 / 3.0 + y2 * (2.0 / 5.0)))
    sp = jnp.maximum(z, 0.0) + lg
    aw = jnp.where(t, 0.25, 0.75)
    return accl + aw * (sig * sig) * sp


def _focal_partial_body(n_seg, lbl_off, preds_hbm, labels_hbm, part_out,
                        pv0, pv1, lv0, lv1, stage_v, sem0, sem1):
    wid = lax.axis_index("s") * _NC + lax.axis_index("c")
    ppt = n_seg // _NW
    nchunks = ppt // _CHUNK
    bufs = ((pv0, lv0, sem0), (pv1, lv1, sem1))

    def start(ci, buf):
        pv, lv, sem = buf
        pbase = wid * ppt + ci * _CHUNK
        handles = []
        for c in range(_NCLS):
            handles.append(pltpu.async_copy(
                preds_hbm.at[pl.ds(c * n_seg + pbase, _CHUNK)],
                pv.at[pl.ds(c * _CHUNK, _CHUNK)], sem))
        handles.append(pltpu.async_copy(
            labels_hbm.at[pl.ds(lbl_off + pbase, _CHUNK)], lv, sem))
        return handles

    pending = {0: start(0, bufs[0])}

    accl = jnp.zeros((_L,), jnp.float32)
    accp = jnp.zeros((_L,), jnp.float32)
    for ci in range(nchunks):
        b = ci % 2
        pv, lv, _ = bufs[b]
        for h in pending.pop(ci):
            h.wait()
        if ci + 1 < nchunks:
            pending[ci + 1] = start(ci + 1, bufs[1 - b])

        def group_body(g, acc, pv=pv, lv=lv):
            accl, accp = acc
            for u in range(_UNROLL):
                off = (g * _UNROLL + u) * _L
                lbl = lv[pl.ds(off, _L)]
                accp = accp + jnp.where(lbl > 0, 1.0, 0.0)
                for c in range(_NCLS):
                    p = pv[pl.ds(c * _CHUNK + off, _L)]
                    accl = _focal_group(p, lbl, c, accl)
            return accl, accp

        accl, accp = lax.fori_loop(0, _CHUNK // (_L * _UNROLL), group_body,
                                   (accl, accp))

    stage_v[pl.ds(0, _L)] = accl
    stage_v[pl.ds(_L, _L)] = accp
    pltpu.sync_copy(stage_v.at[pl.ds(0, _L)],
                    part_out.at[pl.ds(wid * _L, _L)])
    pltpu.sync_copy(stage_v.at[pl.ds(_L, _L)],
                    part_out.at[pl.ds(_NW * _L + wid * _L, _L)])


def _focal_tc_body(p_ref, l_ref, o_loss, o_pos):
    i = pl.program_id(0)

    @pl.when(i == 0)
    def _():
        o_loss[...] = jnp.zeros_like(o_loss)
        o_pos[...] = jnp.zeros_like(o_pos)

    p = p_ref[...]
    lbl = l_ref[...]
    cls = lax.broadcasted_iota(jnp.int32, (_NCLS, _TC_BS), 0) + 1
    t = (lbl == cls).astype(jnp.float32)
    sg = jax.nn.sigmoid(p)
    aw = t * 0.25 + (1.0 - t) * 0.75
    pt = t * (1.0 - sg) + (1.0 - t) * sg
    bce = jnp.maximum(p, 0.0) - p * t + jnp.log1p(jnp.exp(-jnp.abs(p)))
    loss = aw * pt * pt * bce
    o_loss[...] = o_loss[...] + jnp.sum(loss).reshape(1, 1)
    o_pos[...] = o_pos[...] + jnp.sum((lbl > 0).astype(jnp.float32)).reshape(1, 1)


def _reduce_tc_body(*refs):
    sc_parts = refs[:_NSEG]
    tc_loss, tc_pos, o_ref = refs[_NSEG], refs[_NSEG + 1], refs[_NSEG + 2]
    s = tc_loss[0, 0]
    q = tc_pos[0, 0]
    for p_ref in sc_parts:
        part = p_ref[...]
        s = s + jnp.sum(part[:4, :])
        q = q + jnp.sum(part[4:, :])
    o_ref[...] = jnp.reshape(s / jnp.maximum(q, 1.0), (1, 1))


@functools.lru_cache(maxsize=None)
def _build(n_points):
    n_sc = n_points * _SC_FRAC_NUM // _SC_FRAC_DEN
    n_seg = n_sc // _NSEG
    n_tc = n_points - n_sc
    mesh = plsc.VectorSubcoreMesh(core_axis_name="c", subcore_axis_name="s")
    seg_fns = []
    for s in range(_NSEG):
        seg_fns.append(functools.partial(
            pl.kernel,
            mesh=mesh,
            out_type=jax.ShapeDtypeStruct((2 * _NW * _L,), jnp.float32),
            scratch_types=[
                pltpu.VMEM((_CHUNK * _NCLS,), jnp.float32),
                pltpu.VMEM((_CHUNK * _NCLS,), jnp.float32),
                pltpu.VMEM((_CHUNK,), jnp.int32),
                pltpu.VMEM((_CHUNK,), jnp.int32),
                pltpu.VMEM((2 * _L,), jnp.float32),
                pltpu.SemaphoreType.DMA,
                pltpu.SemaphoreType.DMA,
            ],
        )(functools.partial(_focal_partial_body, n_seg, s * n_seg)))

    nb = n_tc // _TC_BS
    off = n_sc // _TC_BS
    tc_fn = pl.pallas_call(
        _focal_tc_body,
        grid=(nb,),
        in_specs=[
            pl.BlockSpec((_NCLS, _TC_BS), lambda i: (0, off + i)),
            pl.BlockSpec((1, _TC_BS), lambda i: (0, off + i)),
        ],
        out_specs=[
            pl.BlockSpec((1, 1), lambda i: (0, 0)),
            pl.BlockSpec((1, 1), lambda i: (0, 0)),
        ],
        out_shape=[
            jax.ShapeDtypeStruct((1, 1), jnp.float32),
            jax.ShapeDtypeStruct((1, 1), jnp.float32),
        ],
    )
    reduce_fn = pl.pallas_call(
        _reduce_tc_body,
        out_shape=jax.ShapeDtypeStruct((1, 1), jnp.float32),
    )
    return seg_fns, tc_fn, reduce_fn


@jax.jit
def kernel(point_cls_preds, point_cls_labels):
    n = point_cls_labels.shape[0]
    seg_fns, tc_fn, reduce_fn = _build(n)
    n_seg = (n * _SC_FRAC_NUM // _SC_FRAC_DEN) // _NSEG
    preds_t = point_cls_preds.T
    parts = []
    for s, fn in enumerate(seg_fns):
        seg = preds_t[:, s * n_seg:(s + 1) * n_seg].reshape(-1)
        parts.append(fn(seg, point_cls_labels))
    tc_loss, tc_pos = tc_fn(preds_t, point_cls_labels.reshape(1, -1))
    out = reduce_fn(*([p.reshape(8, _NW * _L // 4) for p in parts]
                      + [tc_loss, tc_pos]))
    return out[0, 0]


# speedup vs baseline: 1.3987x; 1.1270x over previous
import functools

import jax
import jax.numpy as jnp
from jax import lax
from jax.experimental import pallas as pl
from jax.experimental.pallas import tpu as pltpu
from jax.experimental.pallas import tpu_sc as plsc

_NCLS = 3
_NC = 2
_NS = 16
_NW = _NC * _NS
_L = 16
_NSEG = 2
_CHUNK = 2048
_UNROLL = 2
_SC_FRAC_NUM, _SC_FRAC_DEN = 5, 8
_TC_BS = 16384


def _focal_group(p, lbl, c, accl):
    t = lbl == (c + 1)
    z = jnp.where(t, -p, p)
    e = jnp.exp(-jnp.abs(p))
    inv = 1.0 / (1.0 + e)
    sig = jnp.where(z >= 0, inv, e * inv)
    y = e / (2.0 + e)
    y2 = y * y
    lg = y * (2.0 + y2 * (2.0 / 3.0 + y2 * (2.0 / 5.0)))
    sp = jnp.maximum(z, 0.0) + lg
    aw = jnp.where(t, 0.25, 0.75)
    return accl + aw * (sig * sig) * sp


def _focal_partial_body(n_seg, lbl_off, preds_hbm, labels_hbm, part_out,
                        pv0, pv1, lv0, lv1, stage_v, sem0, sem1):
    wid = lax.axis_index("s") * _NC + lax.axis_index("c")
    ppt = n_seg // _NW
    nchunks = ppt // _CHUNK
    bufs = ((pv0, lv0, sem0), (pv1, lv1, sem1))

    def start(ci, buf):
        pv, lv, sem = buf
        pbase = wid * ppt + ci * _CHUNK
        handles = []
        for c in range(_NCLS):
            handles.append(pltpu.async_copy(
                preds_hbm.at[pl.ds(c * n_seg + pbase, _CHUNK)],
                pv.at[pl.ds(c * _CHUNK, _CHUNK)], sem))
        handles.append(pltpu.async_copy(
            labels_hbm.at[pl.ds(lbl_off + pbase, _CHUNK)], lv, sem))
        return handles

    pending = {0: start(0, bufs[0])}

    accl = jnp.zeros((_L,), jnp.float32)
    accp = jnp.zeros((_L,), jnp.float32)
    for ci in range(nchunks):
        b = ci % 2
        pv, lv, _ = bufs[b]
        for h in pending.pop(ci):
            h.wait()
        if ci + 1 < nchunks:
            pending[ci + 1] = start(ci + 1, bufs[1 - b])

        def group_body(g, acc, pv=pv, lv=lv):
            accl, accp = acc
            for u in range(_UNROLL):
                off = (g * _UNROLL + u) * _L
                lbl = lv[pl.ds(off, _L)]
                accp = accp + jnp.where(lbl > 0, 1.0, 0.0)
                for c in range(_NCLS):
                    p = pv[pl.ds(c * _CHUNK + off, _L)]
                    accl = _focal_group(p, lbl, c, accl)
            return accl, accp

        accl, accp = lax.fori_loop(0, _CHUNK // (_L * _UNROLL), group_body,
                                   (accl, accp))

    stage_v[pl.ds(0, _L)] = accl
    stage_v[pl.ds(_L, _L)] = accp
    pltpu.sync_copy(stage_v.at[pl.ds(0, _L)],
                    part_out.at[pl.ds(wid * _L, _L)])
    pltpu.sync_copy(stage_v.at[pl.ds(_L, _L)],
                    part_out.at[pl.ds(_NW * _L + wid * _L, _L)])


def _focal_tc_body(p_ref, l_ref, o_loss, o_pos):
    i = pl.program_id(0)

    @pl.when(i == 0)
    def _():
        o_loss[...] = jnp.zeros_like(o_loss)
        o_pos[...] = jnp.zeros_like(o_pos)

    p = p_ref[...]
    lbl = l_ref[...]
    cls = lax.broadcasted_iota(jnp.int32, (_NCLS, _TC_BS), 0) + 1
    t = (lbl == cls).astype(jnp.float32)
    sg = jax.nn.sigmoid(p)
    aw = t * 0.25 + (1.0 - t) * 0.75
    pt = t * (1.0 - sg) + (1.0 - t) * sg
    bce = jnp.maximum(p, 0.0) - p * t + jnp.log1p(jnp.exp(-jnp.abs(p)))
    loss = aw * pt * pt * bce
    o_loss[...] = o_loss[...] + jnp.sum(loss).reshape(1, 1)
    o_pos[...] = o_pos[...] + jnp.sum((lbl > 0).astype(jnp.float32)).reshape(1, 1)


def _reduce_tc_body(*refs):
    sc_parts = refs[:_NSEG]
    tc_loss, tc_pos, o_ref = refs[_NSEG], refs[_NSEG + 1], refs[_NSEG + 2]
    s = tc_loss[0, 0]
    q = tc_pos[0, 0]
    for p_ref in sc_parts:
        part = p_ref[...]
        s = s + jnp.sum(part[:4, :])
        q = q + jnp.sum(part[4:, :])
    o_ref[...] = jnp.reshape(s / jnp.maximum(q, 1.0), (1, 1))


@functools.lru_cache(maxsize=None)
def _build(n_points):
    n_sc = n_points * _SC_FRAC_NUM // _SC_FRAC_DEN
    n_seg = n_sc // _NSEG
    n_tc = n_points - n_sc
    mesh = plsc.VectorSubcoreMesh(core_axis_name="c", subcore_axis_name="s")
    seg_fns = []
    for s in range(_NSEG):
        seg_fns.append(functools.partial(
            pl.kernel,
            mesh=mesh,
            out_type=jax.ShapeDtypeStruct((2 * _NW * _L,), jnp.float32),
            scratch_types=[
                pltpu.VMEM((_CHUNK * _NCLS,), jnp.float32),
                pltpu.VMEM((_CHUNK * _NCLS,), jnp.float32),
                pltpu.VMEM((_CHUNK,), jnp.int32),
                pltpu.VMEM((_CHUNK,), jnp.int32),
                pltpu.VMEM((2 * _L,), jnp.float32),
                pltpu.SemaphoreType.DMA,
                pltpu.SemaphoreType.DMA,
            ],
        )(functools.partial(_focal_partial_body, n_seg, s * n_seg)))

    nb = n_tc // _TC_BS
    off = n_sc // _TC_BS
    tc_fn = pl.pallas_call(
        _focal_tc_body,
        grid=(nb,),
        in_specs=[
            pl.BlockSpec((_NCLS, _TC_BS), lambda i: (0, off + i)),
            pl.BlockSpec((1, _TC_BS), lambda i: (0, off + i)),
        ],
        out_specs=[
            pl.BlockSpec((1, 1), lambda i: (0, 0)),
            pl.BlockSpec((1, 1), lambda i: (0, 0)),
        ],
        out_shape=[
            jax.ShapeDtypeStruct((1, 1), jnp.float32),
            jax.ShapeDtypeStruct((1, 1), jnp.float32),
        ],
    )
    reduce_fn = pl.pallas_call(
        _reduce_tc_body,
        out_shape=jax.ShapeDtypeStruct((1, 1), jnp.float32),
    )
    return seg_fns, tc_fn, reduce_fn


@jax.jit
def kernel(point_cls_preds, point_cls_labels):
    n = point_cls_labels.shape[0]
    seg_fns, tc_fn, reduce_fn = _build(n)
    n_seg = (n * _SC_FRAC_NUM // _SC_FRAC_DEN) // _NSEG
    preds_t = point_cls_preds.T
    parts = []
    for s, fn in enumerate(seg_fns):
        seg = preds_t[:, s * n_seg:(s + 1) * n_seg].reshape(-1)
        parts.append(fn(seg, point_cls_labels))
    tc_loss, tc_pos = tc_fn(preds_t, point_cls_labels.reshape(1, -1))
    out = reduce_fn(*([p.reshape(8, _NW * _L // 4) for p in parts]
                      + [tc_loss, tc_pos]))
    return out[0, 0]
